# baseline (speedup 1.0000x reference)
import functools

import jax
import jax.numpy as jnp
from jax import lax
from jax.experimental import pallas as pl
from jax.experimental.pallas import tpu as pltpu
from jax.experimental.pallas import tpu_sc as plsc

NUM_CODEBOOKS = 9
CODEBOOK_SIZE = 1088
EMB_DIM = 64
SEQ_LEN = 131072

NW = 32
LOOKUPS = NUM_CODEBOOKS * SEQ_LEN
SUB = 128
SUBS_PER_CHUNK = 8
STEP = SUBS_PER_CHUNK * SUB
SUBS_PER_HALF = SUBS_PER_CHUNK // 2
HALF = STEP // 2

ROW_W = EMB_DIM // 2

NCH = 4
TCH = SEQ_LEN // NCH
CODE_ROWS_PER_CB = SEQ_LEN // SUB
CHUNK_CODE_ROWS = TCH // SUB


def _sc_gather_chunk(codes2d, tables_flat, chunk):
    mesh = plsc.VectorSubcoreMesh(core_axis_name="c", subcore_axis_name="s")

    @functools.partial(
        pl.kernel,
        mesh=mesh,
        out_type=jax.ShapeDtypeStruct((NUM_CODEBOOKS * TCH, ROW_W),
                                      jnp.float32),
        scratch_types=[
            pltpu.VMEM((SUBS_PER_CHUNK, SUB), jnp.int32),
            pltpu.VMEM((SUBS_PER_CHUNK, SUB), jnp.int32),
            pltpu.VMEM((HALF, ROW_W), jnp.float32),
            pltpu.VMEM((HALF, ROW_W), jnp.float32),
            pltpu.SemaphoreType.DMA,
            pltpu.SemaphoreType.DMA,
        ],
        compiler_params=pltpu.CompilerParams(use_tc_tiling_on_sc=False),
    )
    def k(codes_ref, tables_ref, out_ref, idx_a, idx_b, rows_a, rows_b,
          sem_a, sem_b):
        wid = lax.axis_index("c") * 16 + lax.axis_index("s")

        def load_idx(i, idx_v):
            row = (i * CODE_ROWS_PER_CB + chunk * CHUNK_CODE_ROWS
                   + wid * SUBS_PER_CHUNK)
            pltpu.sync_copy(codes_ref.at[pl.ds(row, SUBS_PER_CHUNK)], idx_v)

        def fire(h, idx_v, rows_v, sem):
            for j in range(SUBS_PER_HALF):
                pltpu.async_copy(tables_ref.at[idx_v.at[SUBS_PER_HALF * h + j]],
                                 rows_v.at[pl.ds(j * SUB, SUB)], sem)

        def drain(rows_v, sem):
            pltpu.make_async_copy(tables_ref.at[pl.ds(0, HALF)],
                                  rows_v, sem).wait()

        def write(i, h, rows_v):
            pltpu.sync_copy(
                rows_v,
                out_ref.at[pl.ds(i * TCH + wid * STEP + h * HALF, HALF)])

        load_idx(0, idx_a)
        fire(0, idx_a, rows_a, sem_a)
        fire(1, idx_a, rows_b, sem_b)

        def pair_body(p, carry):
            a = 2 * p
            b = a + 1
            c = a + 2
            load_idx(b, idx_b)
            drain(rows_a, sem_a)
            write(a, 0, rows_a)
            fire(0, idx_b, rows_a, sem_a)
            drain(rows_b, sem_b)
            write(a, 1, rows_b)
            fire(1, idx_b, rows_b, sem_b)
            load_idx(c, idx_a)
            drain(rows_a, sem_a)
            write(b, 0, rows_a)
            fire(0, idx_a, rows_a, sem_a)
            drain(rows_b, sem_b)
            write(b, 1, rows_b)
            fire(1, idx_a, rows_b, sem_b)
            return carry

        lax.fori_loop(0, (NUM_CODEBOOKS - 1) // 2, pair_body, 0)
        drain(rows_a, sem_a)
        write(NUM_CODEBOOKS - 1, 0, rows_a)
        drain(rows_b, sem_b)
        write(NUM_CODEBOOKS - 1, 1, rows_b)

    return k(codes2d, tables_flat)


BLK = 8192
QB = BLK // 4
H1, H2, H3 = 512, 256, 128


def _mlp_body(e_ref, w1_ref, b1_ref, w2_ref, b2_ref, w3_ref, b3_ref,
              w4_ref, b4_ref, o_ref):
    evens, odds = [], []
    for i in range(NUM_CODEBOOKS):
        w = pltpu.bitcast(e_ref[i], jnp.int32)
        evens.append(pltpu.bitcast(w << 16, jnp.float32))
        odds.append(pltpu.bitcast(w & jnp.int32(-65536), jnp.float32))
    for q in range(4):
        lhs = jnp.concatenate(
            [half[:, ROW_W * q:ROW_W * (q + 1)]
             for i in range(NUM_CODEBOOKS)
             for half in (evens[i], odds[i])],
            axis=1).astype(jnp.bfloat16)
        acc = jnp.dot(lhs, w1_ref[...],
                      preferred_element_type=jnp.float32) + b1_ref[...]
        h = jnp.maximum(acc, 0.0).astype(jnp.bfloat16)
        h = jnp.maximum(
            jnp.dot(h, w2_ref[...], preferred_element_type=jnp.float32)
            + b2_ref[...], 0.0).astype(jnp.bfloat16)
        h = jnp.maximum(
            jnp.dot(h, w3_ref[...], preferred_element_type=jnp.float32)
            + b3_ref[...], 0.0)
        y = jnp.tanh(
            jnp.dot(h, w4_ref[...], preferred_element_type=jnp.float32)
            + b4_ref[...])
        o_ref[:, q] = y[:, 0]


def _tc_mlp(embs, w1, b1, w2, b2, w3, b3, w4, b4, interpret=False):
    grid = (TCH // BLK,)
    return pl.pallas_call(
        _mlp_body,
        grid=grid,
        in_specs=[
            pl.BlockSpec((NUM_CODEBOOKS, QB, 2 * EMB_DIM), lambda j: (0, j, 0)),
            pl.BlockSpec((NUM_CODEBOOKS * EMB_DIM, H1), lambda j: (0, 0)),
            pl.BlockSpec((1, H1), lambda j: (0, 0)),
            pl.BlockSpec((H1, H2), lambda j: (0, 0)),
            pl.BlockSpec((1, H2), lambda j: (0, 0)),
            pl.BlockSpec((H2, H3), lambda j: (0, 0)),
            pl.BlockSpec((1, H3), lambda j: (0, 0)),
            pl.BlockSpec((H3, 1), lambda j: (0, 0)),
            pl.BlockSpec((1, 1), lambda j: (0, 0)),
        ],
        out_specs=pl.BlockSpec((QB, 4), lambda j: (j, 0)),
        out_shape=jax.ShapeDtypeStruct((TCH // 4, 4), jnp.float32),
        interpret=interpret,
    )(embs, w1, b1, w2, b2, w3, b3, w4, b4)


def kernel(audio_codes, tables, W1, b1, W2, b2, W3, b3, W4, b4):
    codes = audio_codes.astype(jnp.int32)
    offs = (jnp.arange(NUM_CODEBOOKS, dtype=jnp.int32) * CODEBOOK_SIZE)[:, None]
    codes2d = (codes + offs).reshape(LOOKUPS // SUB, SUB)
    tables_bf = tables.astype(jnp.bfloat16).reshape(
        NUM_CODEBOOKS * CODEBOOK_SIZE, ROW_W, 2)
    tables_flat = lax.bitcast_convert_type(tables_bf, jnp.float32)
    w1f = W1.astype(jnp.bfloat16).reshape(NUM_CODEBOOKS, EMB_DIM, H1)
    w1 = jnp.concatenate([w1f[:, 0::2, :], w1f[:, 1::2, :]],
                         axis=1).reshape(NUM_CODEBOOKS * EMB_DIM, H1)
    w2 = W2.astype(jnp.bfloat16)
    w3 = W3.astype(jnp.bfloat16)
    b1r, b2r, b3r, b4r = (b1.reshape(1, H1), b2.reshape(1, H2),
                          b3.reshape(1, H3), b4.reshape(1, 1))
    outs = []
    for chunk in range(NCH):
        embs = _sc_gather_chunk(codes2d, tables_flat, chunk).reshape(
            NUM_CODEBOOKS, TCH // 4, 2 * EMB_DIM)
        outs.append(_tc_mlp(embs, w1, b1r, w2, b2r, w3, b3r, W4, b4r))
    return jnp.concatenate(outs, axis=0).reshape(SEQ_LEN)


# speedup vs baseline: 1.0071x; 1.0071x over previous
import functools

import jax
import jax.numpy as jnp
from jax import lax
from jax.experimental import pallas as pl
from jax.experimental.pallas import tpu as pltpu
from jax.experimental.pallas import tpu_sc as plsc

NUM_CODEBOOKS = 9
CODEBOOK_SIZE = 1088
EMB_DIM = 64
SEQ_LEN = 131072

NW = 32
LOOKUPS = NUM_CODEBOOKS * SEQ_LEN
SUB = 128
SUBS_PER_CHUNK = 8
STEP = SUBS_PER_CHUNK * SUB
SUBS_PER_HALF = SUBS_PER_CHUNK // 2
HALF = STEP // 2

ROW_W = EMB_DIM // 2

NCH = 4
TCH = SEQ_LEN // NCH
CODE_ROWS_PER_CB = SEQ_LEN // SUB
CHUNK_CODE_ROWS = TCH // SUB


def _sc_gather_chunk(codes2d, tables_flat, chunk):
    mesh = plsc.VectorSubcoreMesh(core_axis_name="c", subcore_axis_name="s")

    @functools.partial(
        pl.kernel,
        mesh=mesh,
        out_type=jax.ShapeDtypeStruct((NUM_CODEBOOKS * TCH, ROW_W),
                                      jnp.float32),
        scratch_types=[
            pltpu.VMEM((SUBS_PER_CHUNK, SUB), jnp.int32),
            pltpu.VMEM((SUBS_PER_CHUNK, SUB), jnp.int32),
            pltpu.VMEM((HALF, ROW_W), jnp.float32),
            pltpu.VMEM((HALF, ROW_W), jnp.float32),
            pltpu.SemaphoreType.DMA,
            pltpu.SemaphoreType.DMA,
        ],
        compiler_params=pltpu.CompilerParams(use_tc_tiling_on_sc=False),
    )
    def k(codes_ref, tables_ref, out_ref, idx_a, idx_b, rows_a, rows_b,
          sem_a, sem_b):
        wid = lax.axis_index("c") * 16 + lax.axis_index("s")

        def load_idx(i, idx_v):
            row = (i * CODE_ROWS_PER_CB + chunk * CHUNK_CODE_ROWS
                   + wid * SUBS_PER_CHUNK)
            pltpu.sync_copy(codes_ref.at[pl.ds(row, SUBS_PER_CHUNK)], idx_v)

        def fire(h, idx_v, rows_v, sem):
            for j in range(SUBS_PER_HALF):
                pltpu.async_copy(tables_ref.at[idx_v.at[SUBS_PER_HALF * h + j]],
                                 rows_v.at[pl.ds(j * SUB, SUB)], sem)

        def drain(rows_v, sem):
            pltpu.make_async_copy(tables_ref.at[pl.ds(0, HALF)],
                                  rows_v, sem).wait()

        def write(i, h, rows_v):
            pltpu.sync_copy(
                rows_v,
                out_ref.at[pl.ds(i * TCH + wid * STEP + h * HALF, HALF)])

        load_idx(0, idx_a)
        fire(0, idx_a, rows_a, sem_a)
        fire(1, idx_a, rows_b, sem_b)

        def pair_body(p, carry):
            a = 2 * p
            b = a + 1
            c = a + 2
            load_idx(b, idx_b)
            drain(rows_a, sem_a)
            write(a, 0, rows_a)
            fire(0, idx_b, rows_a, sem_a)
            drain(rows_b, sem_b)
            write(a, 1, rows_b)
            fire(1, idx_b, rows_b, sem_b)
            load_idx(c, idx_a)
            drain(rows_a, sem_a)
            write(b, 0, rows_a)
            fire(0, idx_a, rows_a, sem_a)
            drain(rows_b, sem_b)
            write(b, 1, rows_b)
            fire(1, idx_a, rows_b, sem_b)
            return carry

        lax.fori_loop(0, (NUM_CODEBOOKS - 1) // 2, pair_body, 0)
        drain(rows_a, sem_a)
        write(NUM_CODEBOOKS - 1, 0, rows_a)
        drain(rows_b, sem_b)
        write(NUM_CODEBOOKS - 1, 1, rows_b)

    return k(codes2d, tables_flat)


BLK = 4096
QB = BLK // 4
H1, H2, H3 = 512, 256, 128


def _mlp_body(e_ref, w1_ref, b1_ref, w2_ref, b2_ref, w3_ref, b3_ref,
              w4_ref, b4_ref, o_ref):
    evens, odds = [], []
    for i in range(NUM_CODEBOOKS):
        w = pltpu.bitcast(e_ref[i], jnp.int32)
        evens.append(pltpu.bitcast(w << 16, jnp.float32))
        odds.append(pltpu.bitcast(w & jnp.int32(-65536), jnp.float32))
    for q in range(4):
        lhs = jnp.concatenate(
            [half[:, ROW_W * q:ROW_W * (q + 1)]
             for i in range(NUM_CODEBOOKS)
             for half in (evens[i], odds[i])],
            axis=1).astype(jnp.bfloat16)
        acc = jnp.dot(lhs, w1_ref[...],
                      preferred_element_type=jnp.float32) + b1_ref[...]
        h = jnp.maximum(acc, 0.0).astype(jnp.bfloat16)
        h = jnp.maximum(
            jnp.dot(h, w2_ref[...], preferred_element_type=jnp.float32)
            + b2_ref[...], 0.0).astype(jnp.bfloat16)
        h = jnp.maximum(
            jnp.dot(h, w3_ref[...], preferred_element_type=jnp.float32)
            + b3_ref[...], 0.0)
        y = jnp.tanh(
            jnp.dot(h, w4_ref[...], preferred_element_type=jnp.float32)
            + b4_ref[...])
        o_ref[:, q] = y[:, 0]


def _tc_mlp(embs, w1, b1, w2, b2, w3, b3, w4, b4, interpret=False):
    grid = (TCH // BLK,)
    return pl.pallas_call(
        _mlp_body,
        grid=grid,
        in_specs=[
            pl.BlockSpec((NUM_CODEBOOKS, QB, 2 * EMB_DIM), lambda j: (0, j, 0)),
            pl.BlockSpec((NUM_CODEBOOKS * EMB_DIM, H1), lambda j: (0, 0)),
            pl.BlockSpec((1, H1), lambda j: (0, 0)),
            pl.BlockSpec((H1, H2), lambda j: (0, 0)),
            pl.BlockSpec((1, H2), lambda j: (0, 0)),
            pl.BlockSpec((H2, H3), lambda j: (0, 0)),
            pl.BlockSpec((1, H3), lambda j: (0, 0)),
            pl.BlockSpec((H3, 1), lambda j: (0, 0)),
            pl.BlockSpec((1, 1), lambda j: (0, 0)),
        ],
        out_specs=pl.BlockSpec((QB, 4), lambda j: (j, 0)),
        out_shape=jax.ShapeDtypeStruct((TCH // 4, 4), jnp.float32),
        interpret=interpret,
    )(embs, w1, b1, w2, b2, w3, b3, w4, b4)


def kernel(audio_codes, tables, W1, b1, W2, b2, W3, b3, W4, b4):
    codes = audio_codes.astype(jnp.int32)
    offs = (jnp.arange(NUM_CODEBOOKS, dtype=jnp.int32) * CODEBOOK_SIZE)[:, None]
    codes2d = (codes + offs).reshape(LOOKUPS // SUB, SUB)
    tables_bf = tables.astype(jnp.bfloat16).reshape(
        NUM_CODEBOOKS * CODEBOOK_SIZE, ROW_W, 2)
    tables_flat = lax.bitcast_convert_type(tables_bf, jnp.float32)
    w1f = W1.astype(jnp.bfloat16).reshape(NUM_CODEBOOKS, EMB_DIM, H1)
    w1 = jnp.concatenate([w1f[:, 0::2, :], w1f[:, 1::2, :]],
                         axis=1).reshape(NUM_CODEBOOKS * EMB_DIM, H1)
    w2 = W2.astype(jnp.bfloat16)
    w3 = W3.astype(jnp.bfloat16)
    b1r, b2r, b3r, b4r = (b1.reshape(1, H1), b2.reshape(1, H2),
                          b3.reshape(1, H3), b4.reshape(1, 1))
    outs = []
    for chunk in range(NCH):
        embs = _sc_gather_chunk(codes2d, tables_flat, chunk).reshape(
            NUM_CODEBOOKS, TCH // 4, 2 * EMB_DIM)
        outs.append(_tc_mlp(embs, w1, b1r, w2, b2r, w3, b3r, W4, b4r))
    return jnp.concatenate(outs, axis=0).reshape(SEQ_LEN)
